# baseline (speedup 1.0000x reference)
import functools

import jax
import jax.numpy as jnp
import numpy as np
from jax import lax
from jax.experimental import pallas as pl
from jax.experimental.pallas import tpu as pltpu
from jax.experimental.pallas import tpu_sc as plsc

EMB = 128
MARGIN = 0.1
B = 16384
NC = 2
NS = 16
NW = NC * NS
NWA = 8
RPW = B // NWA
CHUNK = 32
NCHUNK = RPW // CHUNK
STRIDE = 17
QSZ = CHUNK * STRIDE

_KS = (2, 3, 2, 2, 2)
_CLS_OFF = []
_o = 0
for _k in _KS:
    _CLS_OFF.append(_o)
    _o += (NCHUNK + 1) * _k * CHUNK
CLS_W = _o
_REL_OFF = (0, (NCHUNK + 1) * CHUNK, 2 * (NCHUNK + 1) * CHUNK)
REL_W = 3 * (NCHUNK + 1) * CHUNK


def _chunked(cols):
    k = len(cols)
    a = jnp.stack(cols, axis=1)
    a = a.reshape(NWA, NCHUNK, CHUNK, k)
    a = a.transpose(0, 1, 3, 2).reshape(NWA, NCHUNK * k * CHUNK)
    pad = jnp.zeros((NWA, k * CHUNK), jnp.int32)
    return jnp.concatenate([a, pad], axis=1)


def _sqrtv(x):
    i = lax.bitcast_convert_type(x, jnp.int32)
    y = lax.bitcast_convert_type(
        np.int32(0x5F3759DF) - lax.shift_right_logical(i, 1), jnp.float32)
    y = y * (1.5 - 0.5 * x * y * y)
    y = y * (1.5 - 0.5 * x * y * y)
    return x * y


def _vec(ref, row, j):
    return ref[row, pl.ds(j * 16, 16)]


def _row_nf1(bc, br, r):
    lb = rt = sh = jnp.zeros((16,), jnp.float32)
    for j in range(8):
        c1 = _vec(bc, r, j)
        c2 = _vec(bc, r, 8 + j)
        d1 = _vec(bc, CHUNK + r, j)
        d2 = _vec(bc, CHUNK + r, 8 + j)
        t = jnp.maximum(d1 - c1 + MARGIN, 0.0)
        lb = lb + t * t
        t = jnp.maximum(c2 - d2 + MARGIN, 0.0)
        rt = rt + t * t
        sh = sh + jnp.maximum(c1 - c2, 0.0) + jnp.maximum(d1 - d2, 0.0)
    return [lb, rt, sh]


def _row_nf2(bc, br, r):
    lb = rt = shc = shd = she = jnp.zeros((16,), jnp.float32)
    for j in range(8):
        c1 = _vec(bc, r, j)
        c2 = _vec(bc, r, 8 + j)
        d1 = _vec(bc, CHUNK + r, j)
        d2 = _vec(bc, CHUNK + r, 8 + j)
        e1 = _vec(bc, 2 * CHUNK + r, j)
        e2 = _vec(bc, 2 * CHUNK + r, 8 + j)
        t = jnp.maximum(e1 - jnp.maximum(c1, d1) + MARGIN, 0.0)
        lb = lb + t * t
        t = jnp.maximum(jnp.minimum(c2, d2) - e2 + MARGIN, 0.0)
        rt = rt + t * t
        t = jnp.maximum(c1 - c2 + MARGIN, 0.0)
        shc = shc + t * t
        t = jnp.maximum(d1 - d2 + MARGIN, 0.0)
        shd = shd + t * t
        t = jnp.maximum(e1 - e2 + MARGIN, 0.0)
        she = she + t * t
    return [lb, rt, shc, shd, she]


def _row_ball(bc, br, r, mode):
    rc = rd = euc = g1 = g2 = jnp.zeros((16,), jnp.float32)
    for j in range(8):
        c1 = _vec(bc, r, j)
        c2 = _vec(bc, r, 8 + j)
        d1 = _vec(bc, CHUNK + r, j)
        d2 = _vec(bc, CHUNK + r, 8 + j)
        rr = _vec(br, r, j)
        t = c2 - c1
        rc = rc + t * t
        t = d2 - d1
        rd = rd + t * t
        if mode == 2:
            x1, x2 = c1, d1
        else:
            x1 = (c1 + c2) * 0.5
            x2 = (d1 + d2) * 0.5
        t = (x1 - rr - x2) if mode == 1 else (x1 + rr - x2)
        euc = euc + t * t
        g1 = g1 + x1 * x1
        g2 = g2 + x2 * x2
    return [rc, rd, euc, g1, g2]


def _fin_nf1(q):
    return _sqrtv(q[0]) + _sqrtv(q[1]) + q[2]


def _fin_nf2(q):
    return (_sqrtv(q[0]) + _sqrtv(q[1]) + _sqrtv(q[2]) + _sqrtv(q[3])
            + _sqrtv(q[4]))


def _reg2(g1, g2):
    return jnp.abs(_sqrtv(g1) - 1.0) + jnp.abs(_sqrtv(g2) - 1.0)


def _fin_nf3(q):
    dst = jnp.maximum(
        _sqrtv(q[2]) + 0.5 * _sqrtv(q[0]) - 0.5 * _sqrtv(q[1]) + MARGIN, 0.0)
    return dst + _reg2(q[3], q[4])


def _fin_nf4(q):
    dst = jnp.maximum(
        _sqrtv(q[2]) - 0.5 * (_sqrtv(q[0]) + _sqrtv(q[1])) - MARGIN, 0.0)
    return dst + _reg2(q[3], q[4])


def _fin_neg(q):
    dst = -(_sqrtv(q[2]) - 0.5 * _sqrtv(q[0]) - 0.5 * _sqrtv(q[1])) + MARGIN
    return dst + _reg2(q[3], q[4])


_TERMS = (
    (2, False, 3, _row_nf1, _fin_nf1),
    (3, False, 5, _row_nf2, _fin_nf2),
    (2, True, 5, functools.partial(_row_ball, mode=0), _fin_nf3),
    (2, True, 5, functools.partial(_row_ball, mode=1), _fin_nf4),
    (2, True, 5, functools.partial(_row_ball, mode=2), _fin_neg),
)


def _sc_body(cw, rw, clsidx, relidx, out,
             b0c, b1c, b0r, b1r, civ, riv, stag, outv, sc0, sc1, sr0, sr1):
    wid = lax.axis_index("s") * NC + lax.axis_index("c")
    act = wid < NWA

    @pl.when(act)
    def _probe():
        _sc_gather_probe(cw, rw, clsidx, relidx, out,
                         b0c, b1c, b0r, b1r, civ, riv, stag, outv,
                         sc0, sc1, sr0, sr1, wid)


def _sc_gather_probe(cw, rw, clsidx, relidx, out,
                     b0c, b1c, b0r, b1r, civ, riv, stag, outv,
                     sc0, sc1, sr0, sr1, wid):
    pltpu.sync_copy(clsidx.at[wid], civ)
    pltpu.sync_copy(relidx.at[wid], riv)
    iota = lax.iota(jnp.int32, 16)

    for ti, (k, has_rel, nq, row_fn, fin_fn) in enumerate(_TERMS[:2]):
        kk = k * CHUNK
        off_c = _CLS_OFF[ti]
        off_r = _REL_OFF[ti - 2] if has_rel else None

        def _cls_copy(c, buf, sem):
            return pltpu.make_async_copy(
                cw.at[civ.at[pl.ds(off_c + c * kk, kk)]],
                buf.at[pl.ds(0, kk)], sem)

        def _rel_copy(c, buf, sem):
            return pltpu.make_async_copy(
                rw.at[riv.at[pl.ds(off_r + c * CHUNK, CHUNK)]], buf, sem)

        def issue(c, bc, br, semc, semr):
            _cls_copy(c, bc, semc).start()
            if has_rel:
                _rel_copy(c, br, semr).start()

        def wait(c, bc, br, semc, semr):
            _cls_copy(c, bc, semc).wait()
            if has_rel:
                _rel_copy(c, br, semr).wait()

        def process(bc, br, accs):
            def row_body(r, carry):
                vals = row_fn(bc, br, r)
                sidx = r * STRIDE + iota
                for qn, v in enumerate(vals):
                    plsc.store_scatter(stag, [sidx + qn * QSZ], v)
                return carry

            lax.fori_loop(0, CHUNK, row_body, 0, unroll=2)

            def jj_body(jj, carry):
                res = []
                for qn in range(nq):
                    for g in range(2):
                        gidx = (g * 16 + iota) * STRIDE + jj + qn * QSZ
                        res.append(carry[qn * 2 + g]
                                   + plsc.load_gather(stag, [gidx]))
                return tuple(res)

            qsums = lax.fori_loop(
                0, 16, jj_body,
                tuple(jnp.zeros((16,), jnp.float32) for _ in range(nq * 2)),
                unroll=2)
            for g in range(2):
                accs = accs + fin_fn([qsums[qn * 2 + g] for qn in range(nq)])
            return accs

        issue(0, b0c, b0r, sc0, sr0)

        def chunk_body(t, accs):
            c0 = 2 * t
            issue(c0 + 1, b1c, b1r, sc1, sr1)
            wait(c0, b0c, b0r, sc0, sr0)
            accs = accs + b0c[0, pl.ds(0, 16)]
            issue(c0 + 2, b0c, b0r, sc0, sr0)
            wait(c0 + 1, b1c, b1r, sc1, sr1)
            accs = accs + b1c[0, pl.ds(0, 16)]
            return accs

        acc = lax.fori_loop(0, NCHUNK // 2, chunk_body,
                            jnp.zeros((16,), jnp.float32))
        wait(NCHUNK, b0c, b0r, sc0, sr0)
        outv[pl.ds(ti * 16, 16)] = acc

    pltpu.sync_copy(outv, out.at[wid])


@functools.partial(
    pl.kernel,
    out_type=jax.ShapeDtypeStruct((NWA, 80), jnp.float32),
    mesh=plsc.VectorSubcoreMesh(core_axis_name="c", subcore_axis_name="s"),
    compiler_params=pltpu.CompilerParams(needs_layout_passes=False),
    scratch_types=[
        pltpu.VMEM((3 * CHUNK, 2 * EMB), jnp.float32),
        pltpu.VMEM((3 * CHUNK, 2 * EMB), jnp.float32),
        pltpu.VMEM((CHUNK, EMB), jnp.float32),
        pltpu.VMEM((CHUNK, EMB), jnp.float32),
        pltpu.VMEM((CLS_W,), jnp.int32),
        pltpu.VMEM((REL_W,), jnp.int32),
        pltpu.VMEM((5 * QSZ,), jnp.float32),
        pltpu.VMEM((80,), jnp.float32),
        pltpu.SemaphoreType.DMA,
        pltpu.SemaphoreType.DMA,
        pltpu.SemaphoreType.DMA,
        pltpu.SemaphoreType.DMA,
    ],
)
def _sc_loss(cw, rw, clsidx, relidx, out, *rest):
    _sc_body(cw, rw, clsidx, relidx, out, *rest)


def kernel(class_emb, rel_emb, nf1_data, nf2_data, nf3_data, nf4_data,
           neg_data):
    i32 = lambda x: x.astype(jnp.int32)
    nf1_data, nf2_data, nf3_data, nf4_data, neg_data = map(
        i32, (nf1_data, nf2_data, nf3_data, nf4_data, neg_data))
    cls = jnp.concatenate([
        _chunked([nf1_data[:, 0], nf1_data[:, 1]]),
        _chunked([nf2_data[:, 0], nf2_data[:, 1], nf2_data[:, 2]]),
        _chunked([nf3_data[:, 0], nf3_data[:, 2]]),
        _chunked([nf4_data[:, 1], nf4_data[:, 2]]),
        _chunked([neg_data[:, 0], neg_data[:, 2]]),
    ], axis=1)
    rel = jnp.concatenate([
        _chunked([nf3_data[:, 1]]),
        _chunked([nf4_data[:, 0]]),
        _chunked([neg_data[:, 1]]),
    ], axis=1)
    partials = _sc_loss(class_emb, rel_emb, cls, rel)
    return jnp.sum(partials) / B


# speedup vs baseline: 1.1624x; 1.1624x over previous
import functools

import jax
import jax.numpy as jnp
import numpy as np
from jax import lax
from jax.experimental import pallas as pl
from jax.experimental.pallas import tpu as pltpu
from jax.experimental.pallas import tpu_sc as plsc

EMB = 128
MARGIN = 0.1
B = 16384
NC = 2
NS = 16
NW = NC * NS
NG = 16
NB = 16
CHUNK = 32
RPG = B // NG
NCH = RPG // CHUNK
BCH = 1024 // 16
STRIDE = 17
QSZ = CHUNK * STRIDE

_NF_KS = (2, 3)
_NF_OFF = (0, (NCH + 1) * _NF_KS[0] * CHUNK)
_NF_SZ = tuple((NCH + 1) * k * CHUNK for k in _NF_KS)
CLS_W = _NF_OFF[1] + _NF_SZ[1]


def _chunked_nf(cols):
    k = len(cols)
    a = jnp.stack(cols, axis=1).reshape(NG, NCH, CHUNK, k)
    a = a.transpose(0, 1, 3, 2).reshape(NG, NCH * k * CHUNK)
    pad = jnp.arange(NG * k * CHUNK, dtype=jnp.int32).reshape(NG, k * CHUNK)
    return jnp.concatenate([a, pad % 100000], axis=1)


def _sqrtv(x):
    i = lax.bitcast_convert_type(x, jnp.int32)
    y = lax.bitcast_convert_type(
        np.int32(0x5F3759DF) - lax.shift_right_logical(i, 1), jnp.float32)
    y = y * (1.5 - 0.5 * x * y * y)
    y = y * (1.5 - 0.5 * x * y * y)
    return x * y


def _vec(ref, row, j):
    return ref[row, pl.ds(j * 16, 16)]


def _row_nf1(bc, r):
    lb = rt = sh = jnp.zeros((16,), jnp.float32)
    for j in range(8):
        c1 = _vec(bc, r, j)
        c2 = _vec(bc, r, 8 + j)
        d1 = _vec(bc, CHUNK + r, j)
        d2 = _vec(bc, CHUNK + r, 8 + j)
        t = jnp.maximum(d1 - c1 + MARGIN, 0.0)
        lb = lb + t * t
        t = jnp.maximum(c2 - d2 + MARGIN, 0.0)
        rt = rt + t * t
        sh = sh + jnp.maximum(c1 - c2, 0.0) + jnp.maximum(d1 - d2, 0.0)
    return [lb, rt, sh]


def _row_nf2(bc, r):
    lb = rt = shc = shd = she = jnp.zeros((16,), jnp.float32)
    for j in range(8):
        c1 = _vec(bc, r, j)
        c2 = _vec(bc, r, 8 + j)
        d1 = _vec(bc, CHUNK + r, j)
        d2 = _vec(bc, CHUNK + r, 8 + j)
        e1 = _vec(bc, 2 * CHUNK + r, j)
        e2 = _vec(bc, 2 * CHUNK + r, 8 + j)
        t = jnp.maximum(e1 - jnp.maximum(c1, d1) + MARGIN, 0.0)
        lb = lb + t * t
        t = jnp.maximum(jnp.minimum(c2, d2) - e2 + MARGIN, 0.0)
        rt = rt + t * t
        t = jnp.maximum(c1 - c2 + MARGIN, 0.0)
        shc = shc + t * t
        t = jnp.maximum(d1 - d2 + MARGIN, 0.0)
        shd = shd + t * t
        t = jnp.maximum(e1 - e2 + MARGIN, 0.0)
        she = she + t * t
    return [lb, rt, shc, shd, she]


def _fin_nf1(q):
    return _sqrtv(q[0]) + _sqrtv(q[1]) + q[2]


def _fin_nf2(q):
    return (_sqrtv(q[0]) + _sqrtv(q[1]) + _sqrtv(q[2]) + _sqrtv(q[3])
            + _sqrtv(q[4]))


def _reg2(g1, g2):
    return jnp.abs(_sqrtv(g1) - 1.0) + jnp.abs(_sqrtv(g2) - 1.0)


def _fin_nf3(q):
    dst = jnp.maximum(
        _sqrtv(q[2]) + 0.5 * _sqrtv(q[0]) - 0.5 * _sqrtv(q[1]) + MARGIN, 0.0)
    return dst + _reg2(q[3], q[4])


def _fin_nf4(q):
    dst = jnp.maximum(
        _sqrtv(q[2]) - 0.5 * (_sqrtv(q[0]) + _sqrtv(q[1])) - MARGIN, 0.0)
    return dst + _reg2(q[3], q[4])


def _fin_neg(q):
    dst = -(_sqrtv(q[2]) - 0.5 * _sqrtv(q[0]) - 0.5 * _sqrtv(q[1])) + MARGIN
    return dst + _reg2(q[3], q[4])


_MASK_HI = np.int32(np.uint32(0xFFFF0000).view(np.int32))


def _bf16_pair(w):
    lo = lax.bitcast_convert_type(lax.shift_left(w, 16), jnp.float32)
    hi = lax.bitcast_convert_type(lax.bitwise_and(w, _MASK_HI), jnp.float32)
    return lo, hi


def _nf_role(cw, cls12, wid, b0c, b1c, civ12, stag, outv, sc0, sc1, iota):
    for ti, (k, row_fn, fin_fn, nq) in enumerate((
            (2, _row_nf1, _fin_nf1, 3), (3, _row_nf2, _fin_nf2, 5))):
        kk = k * CHUNK
        civ = civ12[ti]
        pltpu.sync_copy(cls12[ti].at[wid], civ)

        def _cls_copy(c, buf, sem):
            return pltpu.make_async_copy(
                cw.at[civ.at[pl.ds(c * kk, kk)]], buf.at[pl.ds(0, kk)], sem)

        def process(bc, accs):
            def row_body(r, carry):
                vals = row_fn(bc, r)
                sidx = r * STRIDE + iota
                for qn, v in enumerate(vals):
                    plsc.store_scatter(stag, [sidx + qn * QSZ], v)
                return carry

            lax.fori_loop(0, CHUNK, row_body, 0, unroll=2)

            def jj_body(jj, carry):
                res = []
                for qn in range(nq):
                    for g in range(2):
                        gidx = (g * 16 + iota) * STRIDE + jj + qn * QSZ
                        res.append(carry[qn * 2 + g]
                                   + plsc.load_gather(stag, [gidx]))
                return tuple(res)

            qsums = lax.fori_loop(
                0, 16, jj_body,
                tuple(jnp.zeros((16,), jnp.float32) for _ in range(nq * 2)),
                unroll=2)
            for g in range(2):
                accs = accs + fin_fn([qsums[qn * 2 + g] for qn in range(nq)])
            return accs

        _cls_copy(0, b0c, sc0).start()

        def chunk_body(t, accs):
            c0 = 2 * t
            _cls_copy(c0 + 1, b1c, sc1).start()
            _cls_copy(c0, b0c, sc0).wait()
            accs = process(b0c, accs)
            _cls_copy(c0 + 2, b0c, sc0).start()
            _cls_copy(c0 + 1, b1c, sc1).wait()
            accs = process(b1c, accs)
            return accs

        acc = lax.fori_loop(0, NCH // 2, chunk_body,
                            jnp.zeros((16,), jnp.float32))
        _cls_copy(NCH, b0c, sc0).wait()
        outv[pl.ds(ti * 16, 16)] = acc


def _ball_role(midP, corP, relP, bidx3, ridx3, w2,
               btab, rb0, rb1, rbp, biv, riv, outv, sr0, sr1, iota):
    pltpu.sync_copy(midP, btab)
    i65 = iota * 65

    for ti, (neg_r, fin_fn) in enumerate((
            (False, _fin_nf3), (True, _fin_nf4), (False, _fin_neg))):
        if ti == 2:
            pltpu.sync_copy(corP, btab)
        pltpu.sync_copy(bidx3[ti].at[w2], biv)
        pltpu.sync_copy(ridx3[ti].at[w2], riv)

        def _r_copy(c, buf, sem):
            return pltpu.make_async_copy(
                relP.at[riv.at[pl.ds(c * 16, 16)]], buf, sem)

        def process(rb, accs):
            acc, cbase = accs

            def fill_body(s, carry):
                sidx = s * 65 + iota
                for j4 in range(4):
                    plsc.store_scatter(
                        rbp, [sidx + j4 * 16], rb[s, pl.ds(j4 * 16, 16)])
                return carry

            lax.fori_loop(0, 16, fill_body, 0, unroll=2)

            iv = biv[pl.ds(cbase, 16)] * 65
            jv = biv[pl.ds(1024 + cbase, 16)] * 65

            def w_body(w, carry):
                euc, g1, g2 = carry
                wx1 = plsc.load_gather(btab, [iv + w])
                wx2 = plsc.load_gather(btab, [jv + w])
                wr = plsc.load_gather(rbp, [i65 + w])
                for x1v, x2v, rv in zip(
                        _bf16_pair(wx1), _bf16_pair(wx2), _bf16_pair(wr)):
                    if neg_r:
                        t = x1v - rv - x2v
                    else:
                        t = x1v + rv - x2v
                    euc = euc + t * t
                    g1 = g1 + x1v * x1v
                    g2 = g2 + x2v * x2v
                return euc, g1, g2

            z = jnp.zeros((16,), jnp.float32)
            euc, g1, g2 = lax.fori_loop(0, 64, w_body, (z, z, z), unroll=2)
            rci = lax.bitcast_convert_type(
                plsc.load_gather(btab, [iv + 64]), jnp.float32)
            rcj = lax.bitcast_convert_type(
                plsc.load_gather(btab, [jv + 64]), jnp.float32)
            acc = acc + fin_fn([rci, rcj, euc, g1, g2])
            return (acc, cbase + 16)

        _r_copy(0, rb0, sr0).start()

        def chunk_body(t, accs):
            c0 = 2 * t
            _r_copy(c0 + 1, rb1, sr1).start()
            _r_copy(c0, rb0, sr0).wait()
            accs = process(rb0, accs)
            _r_copy(c0 + 2, rb0, sr0).start()
            _r_copy(c0 + 1, rb1, sr1).wait()
            accs = process(rb1, accs)
            return accs

        acc, _ = lax.fori_loop(0, BCH // 2, chunk_body,
                               (jnp.zeros((16,), jnp.float32),
                                jnp.int32(0)))
        _r_copy(BCH, rb0, sr0).wait()
        outv[pl.ds((2 + ti) * 16, 16)] = acc


def _sc_body(cw, cls1, cls2, midP, corP, relP, bi3, bi4, bin_, ri3, ri4,
             rin, out,
             b0c, b1c, btab, rb0, rb1, rbp, civ1, civ2, biv, riv, stag,
             outv, sc0, sc1, sr0, sr1):
    wid = lax.axis_index("s") * NC + lax.axis_index("c")
    iota = lax.iota(jnp.int32, 16)
    for qn in range(5):
        outv[pl.ds(qn * 16, 16)] = jnp.zeros((16,), jnp.float32)

    @pl.when(wid < NG)
    def _nf():
        _nf_role(cw, (cls1, cls2), wid, b0c, b1c, (civ1, civ2), stag, outv,
                 sc0, sc1, iota)

    @pl.when(wid >= NG)
    def _ball():
        _ball_role(midP, corP, relP, (bi3, bi4, bin_), (ri3, ri4, rin),
                   wid - NG, btab, rb0, rb1, rbp, biv, riv, outv,
                   sr0, sr1, iota)

    pltpu.sync_copy(outv, out.at[wid])


@functools.partial(
    pl.kernel,
    out_type=jax.ShapeDtypeStruct((NW, 80), jnp.float32),
    mesh=plsc.VectorSubcoreMesh(core_axis_name="c", subcore_axis_name="s"),
    compiler_params=pltpu.CompilerParams(needs_layout_passes=False),
    scratch_types=[
        pltpu.VMEM((3 * CHUNK, 2 * EMB), jnp.float32),
        pltpu.VMEM((3 * CHUNK, 2 * EMB), jnp.float32),
        pltpu.VMEM((65000,), jnp.int32),
        pltpu.VMEM((16, 128), jnp.int32),
        pltpu.VMEM((16, 128), jnp.int32),
        pltpu.VMEM((1040,), jnp.int32),
        pltpu.VMEM((_NF_SZ[0],), jnp.int32),
        pltpu.VMEM((_NF_SZ[1],), jnp.int32),
        pltpu.VMEM((2048,), jnp.int32),
        pltpu.VMEM((1056,), jnp.int32),
        pltpu.VMEM((5 * QSZ,), jnp.float32),
        pltpu.VMEM((80,), jnp.float32),
        pltpu.SemaphoreType.DMA,
        pltpu.SemaphoreType.DMA,
        pltpu.SemaphoreType.DMA,
        pltpu.SemaphoreType.DMA,
    ],
)
def _sc_loss(cw, cls1, cls2, midP, corP, relP, bi3, bi4, bin_, ri3, ri4,
             rin, out, *rest):
    _sc_body(cw, cls1, cls2, midP, corP, relP, bi3, bi4, bin_, ri3, ri4,
             rin, out, *rest)


def _prep_body(cwh_ref, mid_ref, rc2_ref):
    c1 = cwh_ref[:, :EMB]
    c2 = cwh_ref[:, EMB:]
    mid_ref[...] = (c1 + c2) * 0.5
    d = c2 - c1
    rc2_ref[...] = jnp.sum(d * d, axis=1, keepdims=True)


def _prep_tables(cwh):
    return pl.pallas_call(
        _prep_body,
        out_shape=(
            jax.ShapeDtypeStruct((1000, EMB), jnp.float32),
            jax.ShapeDtypeStruct((1000, 1), jnp.float32),
        ),
    )(cwh)


def _pack65(x128, rc2):
    xb = x128.astype(jnp.bfloat16).reshape(1000, 64, 2)
    w = jax.lax.bitcast_convert_type(xb, jnp.int32)
    rcw = jax.lax.bitcast_convert_type(rc2, jnp.int32)
    return jnp.concatenate([w, rcw], axis=1).reshape(65000)


def kernel(class_emb, rel_emb, nf1_data, nf2_data, nf3_data, nf4_data,
           neg_data):
    i32 = lambda x: x.astype(jnp.int32)
    nf1_data, nf2_data, nf3_data, nf4_data, neg_data = map(
        i32, (nf1_data, nf2_data, nf3_data, nf4_data, neg_data))

    cls1 = _chunked_nf([nf1_data[:, 0], nf1_data[:, 1]])
    cls2 = _chunked_nf([nf2_data[:, 0], nf2_data[:, 1], nf2_data[:, 2]])

    def _pair(i_col, j_col):
        return jnp.concatenate(
            [i_col.reshape(NB, 1024), j_col.reshape(NB, 1024)], axis=1)

    bi3 = _pair(nf3_data[:, 0], nf3_data[:, 2])
    bi4 = _pair(nf4_data[:, 1], nf4_data[:, 2])
    bin_ = _pair(neg_data[:, 0], neg_data[:, 2])

    def _rcol(col):
        a = col.reshape(NB, 1024)
        pad = (jnp.arange(NB * 32, dtype=jnp.int32).reshape(NB, 32)) % 1000
        return jnp.concatenate([a, pad], axis=1)

    ri3 = _rcol(nf3_data[:, 1])
    ri4 = _rcol(nf4_data[:, 0])
    rin = _rcol(neg_data[:, 1])

    mid, rc2 = _prep_tables(class_emb[:1000])
    midP = _pack65(mid, rc2)
    corP = _pack65(class_emb[:1000, :EMB], rc2)
    relP = jax.lax.bitcast_convert_type(
        rel_emb.astype(jnp.bfloat16).reshape(1000, 64, 2), jnp.int32)
    relP = jnp.pad(relP, ((0, 0), (0, 64)))

    partials = _sc_loss(class_emb, cls1, cls2, midP, corP, relP,
                        bi3, bi4, bin_, ri3, ri4, rin)
    return jnp.sum(partials) / B
